# Initial kernel scaffold; baseline (speedup 1.0000x reference)
import jax
import jax.numpy as jnp
from jax.experimental import pallas as pl


def kernel(x, pe):
    raise NotImplementedError("write your pallas kernel here")


# speedup vs baseline: 5.0847x; 5.0847x over previous
import functools

import jax
import jax.numpy as jnp
from jax import lax
from jax.experimental import pallas as pl
from jax.experimental.pallas import tpu as pltpu
from jax.experimental.pallas import tpu_sc as plsc

MAX_LEN = 200
D = 64
B = 4096
L = 200
PAD_ROWS = (L - 1) + (MAX_LEN + 1)
NW = 32
ROWS_PER_W = B // NW
N_CHUNKS = (L + 15) // 16


def _clamp01(v):
    return jnp.minimum(jnp.maximum(v, 0), 1)


def _sc_body(x_hbm, tab_hbm, out_hbm, tab_v, x_v, spill_v):
    cid = lax.axis_index("c")
    sid = lax.axis_index("s")
    wid = sid * 2 + cid
    base = wid * ROWS_PER_W

    pltpu.sync_copy(tab_hbm, tab_v)
    pltpu.sync_copy(x_hbm.at[pl.ds(base * L, ROWS_PER_W * L)],
                    x_v.at[pl.ds(0, ROWS_PER_W * L)])

    lane = lax.iota(jnp.int32, 16)

    def row_body(r, carry):
        off = r * L
        cnt = jnp.zeros((16,), jnp.int32)
        for k in range(N_CHUNKS):
            chunk = x_v[pl.ds(off + 16 * k, 16)]
            v01 = jnp.minimum(chunk, 1)
            if 16 * (k + 1) > L:
                v01 = v01 * _clamp01((L - 16 * k) - lane)
            cnt = cnt + v01
        spill_v[pl.ds(0, 16)] = cnt
        cv = spill_v[pl.ds(0, 16)]
        c = cv[0]
        for ln in range(1, 16):
            c = c + cv[ln]

        out_off = (base + r) * (L * D)
        pltpu.sync_copy(tab_v.at[pl.ds(c * D, L * D)],
                        out_hbm.at[pl.ds(out_off, L * D)])

        @pl.when(c < L)
        def _patch():
            def jbody(jc, carry2):
                chunk = x_v[pl.ds(off + 16 * jc, 16)]
                inval = (1 - jnp.minimum(chunk, 1)) * _clamp01(
                    (L - 16 * jc) - lane)
                spill_v[pl.ds(0, 16)] = inval
                iv = spill_v[pl.ds(0, 16)]
                for ln in range(16):
                    @pl.when(iv[ln] != 0)
                    def _():
                        pltpu.sync_copy(
                            tab_v.at[pl.ds(0, D)],
                            out_hbm.at[pl.ds(out_off + (jc * 16 + ln) * D,
                                             D)])
                return carry2
            lax.fori_loop(0, N_CHUNKS, jbody, 0)
        return carry

    lax.fori_loop(0, ROWS_PER_W, row_body, 0)


_sc_call = functools.partial(
    pl.kernel,
    mesh=plsc.VectorSubcoreMesh(core_axis_name="c", subcore_axis_name="s"),
    out_type=jax.ShapeDtypeStruct((B * L * D,), jnp.float32),
    scratch_types=[
        pltpu.VMEM((PAD_ROWS * D,), jnp.float32),
        pltpu.VMEM((ROWS_PER_W * L + 16,), jnp.int32),
        pltpu.VMEM((16,), jnp.int32),
    ],
)(_sc_body)


def kernel(x, pe):
    pad = jnp.concatenate(
        [jnp.broadcast_to(pe[0:1], (L - 1, D)), pe], axis=0)
    out_flat = _sc_call(x.reshape(-1), pad.reshape(-1))
    return out_flat.reshape(B, L, D)


# speedup vs baseline: 5.1272x; 1.0084x over previous
import functools

import jax
import jax.numpy as jnp
from jax import lax
from jax.experimental import pallas as pl
from jax.experimental.pallas import tpu as pltpu
from jax.experimental.pallas import tpu_sc as plsc

MAX_LEN = 200
D = 64
B = 4096
L = 200
PAD_ROWS = (L - 1) + (MAX_LEN + 1)
NW = 32
ROWS_PER_W = B // NW
N_CHUNKS = (L + 15) // 16
WIN = 8


def _clamp01(v):
    return jnp.minimum(jnp.maximum(v, 0), 1)


def _sc_body(x_hbm, tab_hbm, out_hbm, tab_v, x_v, spill_v, cnt_s, sem):
    cid = lax.axis_index("c")
    sid = lax.axis_index("s")
    wid = sid * 2 + cid
    base = wid * ROWS_PER_W

    pltpu.sync_copy(tab_hbm, tab_v)
    pltpu.sync_copy(x_hbm.at[pl.ds(base * L, ROWS_PER_W * L)],
                    x_v.at[pl.ds(0, ROWS_PER_W * L)])

    lane = lax.iota(jnp.int32, 16)

    def fire(r):
        off = r * L
        cnt = jnp.zeros((16,), jnp.int32)
        for k in range(N_CHUNKS):
            chunk = x_v[pl.ds(off + 16 * k, 16)]
            v01 = jnp.minimum(chunk, 1)
            if 16 * (k + 1) > L:
                v01 = v01 * _clamp01((L - 16 * k) - lane)
            cnt = cnt + v01
        spill_v[pl.ds(0, 16)] = cnt
        cv = spill_v[pl.ds(0, 16)]
        c = cv[0]
        for ln in range(1, 16):
            c = c + cv[ln]
        cnt_s[r] = c
        pltpu.async_copy(tab_v.at[pl.ds(c * D, L * D)],
                         out_hbm.at[pl.ds((base + r) * (L * D), L * D)],
                         sem)

    def wait_one():
        pltpu.make_async_copy(
            tab_v.at[pl.ds(0, L * D)],
            out_hbm.at[pl.ds(base * (L * D), L * D)], sem).wait()

    def prologue(r, carry):
        fire(r)
        return carry
    lax.fori_loop(0, WIN, prologue, 0)

    def steady(r, carry):
        wait_one()
        fire(r)
        return carry
    lax.fori_loop(WIN, ROWS_PER_W, steady, 0)

    def drain(i, carry):
        wait_one()
        return carry
    lax.fori_loop(0, WIN, drain, 0)

    def patch_pass(r, carry):
        c = cnt_s[r]

        @pl.when(c < L)
        def _patch():
            off = r * L
            out_off = (base + r) * (L * D)

            def jbody(jc, carry2):
                chunk = x_v[pl.ds(off + 16 * jc, 16)]
                inval = (1 - jnp.minimum(chunk, 1)) * _clamp01(
                    (L - 16 * jc) - lane)
                spill_v[pl.ds(0, 16)] = inval
                iv = spill_v[pl.ds(0, 16)]
                for ln in range(16):
                    @pl.when(iv[ln] != 0)
                    def _():
                        pltpu.sync_copy(
                            tab_v.at[pl.ds(0, D)],
                            out_hbm.at[pl.ds(out_off + (jc * 16 + ln) * D,
                                             D)])
                return carry2
            lax.fori_loop(0, N_CHUNKS, jbody, 0)
        return carry
    lax.fori_loop(0, ROWS_PER_W, patch_pass, 0)


_sc_call = functools.partial(
    pl.kernel,
    mesh=plsc.VectorSubcoreMesh(core_axis_name="c", subcore_axis_name="s"),
    out_type=jax.ShapeDtypeStruct((B * L * D,), jnp.float32),
    scratch_types=[
        pltpu.VMEM((PAD_ROWS * D,), jnp.float32),
        pltpu.VMEM((ROWS_PER_W * L + 16,), jnp.int32),
        pltpu.VMEM((16,), jnp.int32),
        pltpu.SMEM((ROWS_PER_W,), jnp.int32),
        pltpu.SemaphoreType.DMA,
    ],
)(_sc_body)


def kernel(x, pe):
    pad = jnp.concatenate(
        [jnp.broadcast_to(pe[0:1], (L - 1, D)), pe], axis=0)
    out_flat = _sc_call(x.reshape(-1), pad.reshape(-1))
    return out_flat.reshape(B, L, D)


# speedup vs baseline: 19.1035x; 3.7259x over previous
import functools

import jax
import jax.numpy as jnp
from jax import lax
from jax.experimental import pallas as pl
from jax.experimental.pallas import tpu as pltpu
from jax.experimental.pallas import tpu_sc as plsc

MAX_LEN = 200
D = 64
B = 4096
L = 200
PAD_ROWS = (L - 1) + (MAX_LEN + 1)
NW = 32
ROWS_PER_W = B // NW
N_CHUNKS = (L + 15) // 16


def _clamp01(v):
    return jnp.minimum(jnp.maximum(v, 0), 1)


def _sc_body(x_hbm, tab_hbm, out_hbm, tab_v, x_v, spill_v, buf_v,
             cnt_s, dev_s, sem):
    cid = lax.axis_index("c")
    sid = lax.axis_index("s")
    bt = sid * 2 + cid

    pltpu.sync_copy(tab_hbm, tab_v)
    pltpu.sync_copy(x_hbm.at[pl.ds(bt * ROWS_PER_W * L, ROWS_PER_W * L)],
                    x_v.at[pl.ds(0, ROWS_PER_W * L)])

    lane = lax.iota(jnp.int32, 16)

    def count_row(r, ndev):
        off = r * L
        cnt = jnp.zeros((16,), jnp.int32)
        for k in range(N_CHUNKS):
            chunk = x_v[pl.ds(off + 16 * k, 16)]
            v01 = jnp.minimum(chunk, 1)
            if 16 * (k + 1) > L:
                v01 = v01 * _clamp01((L - 16 * k) - lane)
            cnt = cnt + v01
        spill_v[pl.ds(0, 16)] = cnt
        cv = spill_v[pl.ds(0, 16)]
        c = cv[0]
        for ln in range(1, 16):
            c = c + cv[ln]
        cnt_s[r] = c

        @pl.when(c < L)
        def _():
            dev_s[ndev] = r
        return jnp.where(c < L, ndev + 1, ndev)

    ndev = lax.fori_loop(0, ROWS_PER_W, count_row, 0)

    def build_block(j, slot):
        trow = L + j
        for k in range(4):
            chunk = tab_v[pl.ds(trow * D + 16 * k, 16)]
            for ln in range(16):
                d = 16 * k + ln
                vec = jnp.broadcast_to(chunk[ln], (16,))
                for m in range(8):
                    buf_v[slot, d // 8, d % 8, pl.ds(16 * m, 16)] = vec

        def fix_dev(dv, carry):
            r = dev_s[dv]
            cr = cnt_s[r]
            t = lax.bitwise_and(j, 15)
            chunk = x_v[pl.ds(r * L + j - t, 16)]
            em = _clamp01(1 - (lane - t) * (lane - t))
            rv = jnp.minimum(chunk, 1) * em * (cr + j)
            spill_v[pl.ds(0, 16)] = rv
            sv = spill_v[pl.ds(0, 16)]
            rowi = sv[0]
            for ln2 in range(1, 16):
                rowi = rowi + sv[ln2]
            g16 = lax.bitwise_and(r, 127 - 15)
            t2 = lax.bitwise_and(r, 15)
            emf = _clamp01(1 - (lane - t2) * (lane - t2)).astype(jnp.float32)
            kef = 1.0 - emf
            for k in range(4):
                vals = tab_v[pl.ds(rowi * D + 16 * k, 16)]
                for ln2 in range(16):
                    d = 16 * k + ln2
                    vb = jnp.broadcast_to(vals[ln2], (16,))
                    old = buf_v[slot, d // 8, d % 8, pl.ds(g16, 16)]
                    buf_v[slot, d // 8, d % 8, pl.ds(g16, 16)] = (
                        old * kef + vb * emf)
            return carry
        lax.fori_loop(0, ndev, fix_dev, 0)

        for dt in range(8):
            pltpu.async_copy(buf_v.at[slot, dt], out_hbm.at[j, dt, bt], sem)

    def wait_slot():
        for _ in range(8):
            pltpu.make_async_copy(
                buf_v.at[0, 0], out_hbm.at[0, 0, bt], sem).wait()

    def jj_body(jj, carry):
        @pl.when(jj > 0)
        def _():
            wait_slot()
            wait_slot()
        build_block(2 * jj, 0)
        build_block(2 * jj + 1, 1)
        return carry

    lax.fori_loop(0, L // 2, jj_body, 0)
    wait_slot()
    wait_slot()


_sc_call = functools.partial(
    pl.kernel,
    mesh=plsc.VectorSubcoreMesh(core_axis_name="c", subcore_axis_name="s"),
    out_type=jax.ShapeDtypeStruct((L, 8, NW, 8, 128), jnp.float32),
    scratch_types=[
        pltpu.VMEM((PAD_ROWS * D,), jnp.float32),
        pltpu.VMEM((ROWS_PER_W * L + 16,), jnp.int32),
        pltpu.VMEM((16,), jnp.int32),
        pltpu.VMEM((2, 8, 8, 128), jnp.float32),
        pltpu.SMEM((ROWS_PER_W,), jnp.int32),
        pltpu.SMEM((ROWS_PER_W,), jnp.int32),
        pltpu.SemaphoreType.DMA,
    ],
)(_sc_body)


def kernel(x, pe):
    pad = jnp.concatenate(
        [jnp.broadcast_to(pe[0:1], (L - 1, D)), pe], axis=0)
    phys = _sc_call(x.reshape(-1), pad.reshape(-1))
    return phys.transpose(2, 4, 0, 1, 3).reshape(B, L, D)


# speedup vs baseline: 19.2976x; 1.0102x over previous
import functools

import jax
import jax.numpy as jnp
from jax import lax
from jax.experimental import pallas as pl
from jax.experimental.pallas import tpu as pltpu
from jax.experimental.pallas import tpu_sc as plsc

MAX_LEN = 200
D = 64
B = 4096
L = 200
PAD_ROWS = (L - 1) + (MAX_LEN + 1)
NW = 32
ROWS_PER_W = B // NW
N_CHUNKS = (L + 15) // 16


def _clamp01(v):
    return jnp.minimum(jnp.maximum(v, 0), 1)


def _sc_body(x_hbm, tab_hbm, out_hbm, tab_v, x_v, spill_v, buf_v,
             cnt_s, dev_s, sem):
    cid = lax.axis_index("c")
    sid = lax.axis_index("s")
    bt = sid * 2 + cid

    pltpu.sync_copy(tab_hbm, tab_v)
    pltpu.sync_copy(x_hbm.at[pl.ds(bt * ROWS_PER_W * L, ROWS_PER_W * L)],
                    x_v.at[pl.ds(0, ROWS_PER_W * L)])

    lane = lax.iota(jnp.int32, 16)

    def count_row(r, ndev):
        off = r * L
        cnt = jnp.zeros((16,), jnp.int32)
        for k in range(N_CHUNKS):
            chunk = x_v[pl.ds(off + 16 * k, 16)]
            v01 = jnp.minimum(chunk, 1)
            if 16 * (k + 1) > L:
                v01 = v01 * _clamp01((L - 16 * k) - lane)
            cnt = cnt + v01
        spill_v[pl.ds(0, 16)] = cnt
        cv = spill_v[pl.ds(0, 16)]
        c = cv[0]
        for ln in range(1, 16):
            c = c + cv[ln]
        cnt_s[r] = c

        @pl.when(c < L)
        def _():
            dev_s[ndev] = r
        return jnp.where(c < L, ndev + 1, ndev)

    ndev = lax.fori_loop(0, ROWS_PER_W, count_row, 0)

    def build_block(j, slot):
        trow = L + j
        for k in range(4):
            chunk = tab_v[pl.ds(trow * D + 16 * k, 16)]
            for ln in range(16):
                d = 16 * k + ln
                vec = jnp.broadcast_to(chunk[ln], (16,))
                for m in range(8):
                    buf_v[slot, d // 8, d % 8, pl.ds(16 * m, 16)] = vec

        def fix_dev(dv, carry):
            r = dev_s[dv]
            cr = cnt_s[r]
            t = lax.bitwise_and(j, 15)
            chunk = x_v[pl.ds(r * L + j - t, 16)]
            em = _clamp01(1 - (lane - t) * (lane - t))
            rv = jnp.minimum(chunk, 1) * em * (cr + j)
            spill_v[pl.ds(0, 16)] = rv
            sv = spill_v[pl.ds(0, 16)]
            rowi = sv[0]
            for ln2 in range(1, 16):
                rowi = rowi + sv[ln2]
            g16 = lax.bitwise_and(r, 127 - 15)
            t2 = lax.bitwise_and(r, 15)
            emf = _clamp01(1 - (lane - t2) * (lane - t2)).astype(jnp.float32)
            kef = 1.0 - emf
            for k in range(4):
                vals = tab_v[pl.ds(rowi * D + 16 * k, 16)]
                for ln2 in range(16):
                    d = 16 * k + ln2
                    vb = jnp.broadcast_to(vals[ln2], (16,))
                    old = buf_v[slot, d // 8, d % 8, pl.ds(g16, 16)]
                    buf_v[slot, d // 8, d % 8, pl.ds(g16, 16)] = (
                        old * kef + vb * emf)
            return carry
        lax.fori_loop(0, ndev, fix_dev, 0)

        pltpu.async_copy(buf_v.at[slot], out_hbm.at[j, :, bt], sem)

    def wait_slot():
        pltpu.make_async_copy(
            buf_v.at[0], out_hbm.at[0, :, bt], sem).wait()

    def jj_body(jj, carry):
        @pl.when(jj > 0)
        def _():
            wait_slot()
            wait_slot()
        build_block(2 * jj, 0)
        build_block(2 * jj + 1, 1)
        return carry

    lax.fori_loop(0, L // 2, jj_body, 0)
    wait_slot()
    wait_slot()


_sc_call = functools.partial(
    pl.kernel,
    mesh=plsc.VectorSubcoreMesh(core_axis_name="c", subcore_axis_name="s"),
    out_type=jax.ShapeDtypeStruct((L, 8, NW, 8, 128), jnp.float32),
    scratch_types=[
        pltpu.VMEM((PAD_ROWS * D,), jnp.float32),
        pltpu.VMEM((ROWS_PER_W * L + 16,), jnp.int32),
        pltpu.VMEM((16,), jnp.int32),
        pltpu.VMEM((2, 8, 8, 128), jnp.float32),
        pltpu.SMEM((ROWS_PER_W,), jnp.int32),
        pltpu.SMEM((ROWS_PER_W,), jnp.int32),
        pltpu.SemaphoreType.DMA,
    ],
)(_sc_body)


def kernel(x, pe):
    pad = jnp.concatenate(
        [jnp.broadcast_to(pe[0:1], (L - 1, D)), pe], axis=0)
    phys = _sc_call(x.reshape(-1), pad.reshape(-1))
    return phys.transpose(2, 4, 0, 1, 3).reshape(B, L, D)


# speedup vs baseline: 19.6641x; 1.0190x over previous
import functools

import jax
import jax.numpy as jnp
from jax import lax
from jax.experimental import pallas as pl
from jax.experimental.pallas import tpu as pltpu
from jax.experimental.pallas import tpu_sc as plsc

MAX_LEN = 200
D = 64
B = 4096
L = 200
PAD_ROWS = (L - 1) + (MAX_LEN + 1)
NW = 32
ROWS_PER_W = B // NW


def _clamp01(v):
    return jnp.minimum(jnp.maximum(v, 0), 1)


def _sc_body(xt_hbm, tab_hbm, out_hbm, tab_v, x_v, ct_v, spill_v, buf_v,
             cnt_s, dev_s, sem):
    cid = lax.axis_index("c")
    sid = lax.axis_index("s")
    bt = sid * 2 + cid

    pltpu.sync_copy(tab_hbm, tab_v)
    pltpu.sync_copy(xt_hbm.at[:, pl.ds(bt * ROWS_PER_W, ROWS_PER_W)], x_v)

    lane = lax.iota(jnp.int32, 16)

    for g in range(8):
        ct_v[pl.ds(16 * g, 16)] = jnp.zeros((16,), jnp.int32)

    def count_j(j, carry):
        for g in range(8):
            chunk = x_v[j, pl.ds(16 * g, 16)]
            ct_v[pl.ds(16 * g, 16)] = (
                ct_v[pl.ds(16 * g, 16)] + jnp.minimum(chunk, 1))
        return carry
    lax.fori_loop(0, L, count_j, 0)

    def scan_row(r, ndev):
        g16 = lax.bitwise_and(r, 112)
        t = lax.bitwise_and(r, 15)
        em = _clamp01(1 - (lane - t) * (lane - t))
        spill_v[pl.ds(0, 16)] = ct_v[pl.ds(g16, 16)] * em
        sv = spill_v[pl.ds(0, 16)]
        c = sv[0]
        for ln in range(1, 16):
            c = c + sv[ln]
        cnt_s[r] = c

        @pl.when(c < L)
        def _():
            dev_s[ndev] = r
        return jnp.where(c < L, ndev + 1, ndev)

    ndev = lax.fori_loop(0, ROWS_PER_W, scan_row, 0)

    def build_block(j, slot):
        trow = L + j
        for k in range(4):
            chunk = tab_v[pl.ds(trow * D + 16 * k, 16)]
            for ln in range(16):
                d = 16 * k + ln
                vec = jnp.broadcast_to(chunk[ln], (16,))
                for m in range(8):
                    buf_v[slot, d // 8, d % 8, pl.ds(16 * m, 16)] = vec

        def fix_dev(dv, carry):
            r = dev_s[dv]
            cr = cnt_s[r]
            g16 = lax.bitwise_and(r, 112)
            t = lax.bitwise_and(r, 15)
            em = _clamp01(1 - (lane - t) * (lane - t))
            xg = x_v[j, pl.ds(g16, 16)]
            spill_v[pl.ds(0, 16)] = jnp.minimum(xg, 1) * em
            sv = spill_v[pl.ds(0, 16)]
            s01 = sv[0]
            for ln in range(1, 16):
                s01 = s01 + sv[ln]
            rowi = s01 * (cr + j)
            emf = em.astype(jnp.float32)
            kef = 1.0 - emf
            for k in range(4):
                vals = tab_v[pl.ds(rowi * D + 16 * k, 16)]
                for ln2 in range(16):
                    d = 16 * k + ln2
                    vb = jnp.broadcast_to(vals[ln2], (16,))
                    old = buf_v[slot, d // 8, d % 8, pl.ds(g16, 16)]
                    buf_v[slot, d // 8, d % 8, pl.ds(g16, 16)] = (
                        old * kef + vb * emf)
            return carry
        lax.fori_loop(0, ndev, fix_dev, 0)

        pltpu.async_copy(buf_v.at[slot], out_hbm.at[j, :, bt], sem)

    def wait_slot():
        pltpu.make_async_copy(
            buf_v.at[0], out_hbm.at[0, :, bt], sem).wait()

    def jj_body(jj, carry):
        @pl.when(jj > 0)
        def _():
            wait_slot()
            wait_slot()
        build_block(2 * jj, 0)
        build_block(2 * jj + 1, 1)
        return carry

    lax.fori_loop(0, L // 2, jj_body, 0)
    wait_slot()
    wait_slot()


_sc_call = functools.partial(
    pl.kernel,
    mesh=plsc.VectorSubcoreMesh(core_axis_name="c", subcore_axis_name="s"),
    out_type=jax.ShapeDtypeStruct((L, 8, NW, 8, 128), jnp.float32),
    scratch_types=[
        pltpu.VMEM((PAD_ROWS * D,), jnp.float32),
        pltpu.VMEM((L, ROWS_PER_W), jnp.int32),
        pltpu.VMEM((ROWS_PER_W,), jnp.int32),
        pltpu.VMEM((16,), jnp.int32),
        pltpu.VMEM((2, 8, 8, 128), jnp.float32),
        pltpu.SMEM((ROWS_PER_W,), jnp.int32),
        pltpu.SMEM((ROWS_PER_W,), jnp.int32),
        pltpu.SemaphoreType.DMA,
    ],
)(_sc_body)


def kernel(x, pe):
    pad = jnp.concatenate(
        [jnp.broadcast_to(pe[0:1], (L - 1, D)), pe], axis=0)
    phys = _sc_call(jnp.swapaxes(x, 0, 1), pad.reshape(-1))
    return phys.transpose(2, 4, 0, 1, 3).reshape(B, L, D)
